# baseline (speedup 1.0000x reference)
import functools

import jax
import jax.numpy as jnp
from jax import lax
from jax.experimental import pallas as pl
from jax.experimental.pallas import tpu as pltpu
from jax.experimental.pallas import tpu_sc as plsc

_INPUT_DIM = 3
_PARTITION_NUM = 100
_OUTPUT_DIM = 16
_B = 16384

_info = plsc.get_sparse_core_info()
_NC, _NS, _L = _info.num_cores, _info.num_subcores, _info.num_lanes
_NW = _NC * _NS
_B_PER_W = _B // _NW


def _body(inputs_hbm, table_hbm, out_hbm, chunk_v, idx_v, off_v, val_v, sem):
    wid = lax.axis_index("s") * _NC + lax.axis_index("c")
    base = wid * _B_PER_W

    pltpu.sync_copy(
        inputs_hbm.at[pl.ds(base * _INPUT_DIM, _B_PER_W * _INPUT_DIM)],
        chunk_v)

    lane3 = lax.iota(jnp.int32, _L) * _INPUT_DIM
    for t in range(_B_PER_W // _L):
        digits = []
        for d in range(_INPUT_DIM):
            x = plsc.load_gather(chunk_v, [lane3 + (t * _L * _INPUT_DIM + d)])
            x = jnp.maximum(x, 0.0)
            s = (x * jnp.float32(_PARTITION_NUM)).astype(jnp.int32)
            digits.append(jnp.minimum(s, _PARTITION_NUM - 1))
        idx = digits[0] + digits[1] * _PARTITION_NUM \
            + digits[2] * (_PARTITION_NUM * _PARTITION_NUM)
        idx_v[pl.ds(t * _L, _L)] = idx * _OUTPUT_DIM

    copies = []
    for j in range(_OUTPUT_DIM):
        for t in range(_B_PER_W // _L):
            off_v[j][pl.ds(t * _L, _L)] = idx_v[pl.ds(t * _L, _L)] + j
        copies.append(pltpu.async_copy(table_hbm.at[off_v[j]], val_v[j], sem))
    for j in range(_OUTPUT_DIM):
        copies[j].wait()
        pltpu.sync_copy(val_v[j], out_hbm.at[j].at[pl.ds(base, _B_PER_W)])


@jax.jit
def kernel(inputs, table):
    mesh = plsc.VectorSubcoreMesh(core_axis_name="c", subcore_axis_name="s")
    fn = pl.kernel(
        _body,
        mesh=mesh,
        compiler_params=pltpu.CompilerParams(use_tc_tiling_on_sc=False,
                                             needs_layout_passes=False),
        out_type=jax.ShapeDtypeStruct((_OUTPUT_DIM, _B), jnp.float32),
        scratch_types=[
            pltpu.VMEM((_B_PER_W * _INPUT_DIM,), jnp.float32),
            pltpu.VMEM((_B_PER_W,), jnp.int32),
            [pltpu.VMEM((_B_PER_W,), jnp.int32)] * _OUTPUT_DIM,
            [pltpu.VMEM((_B_PER_W,), jnp.float32)] * _OUTPUT_DIM,
            pltpu.SemaphoreType.DMA,
        ],
    )
    out_t = fn(inputs.reshape(-1), table.reshape(-1))
    return out_t.T
